# baseline (speedup 1.0000x reference)
import functools

import jax
import jax.numpy as jnp
from jax import lax
from jax.experimental import pallas as pl
from jax.experimental.pallas import tpu as pltpu
from jax.experimental.pallas import tpu_sc as plsc


N = 10000
E = 320000
D = 128
H = 64
P = 32
G = 128

NP = 10240
BLK = 1024
NSTEPS = NP // BLK

NC = 2
NS = 16
NTILES = NC * NS
EPT = E // NTILES
K = 32
NB = 8
WAVE = NB // 2
C = 320
EPT_PAD = C * K
RPT = NP // NS


def _fill_rows(ref, nrows, ncols, value):
  vec = jnp.full((16,), value, jnp.float32)
  per_row = ncols // 16

  def body(j, carry):
    r = j // per_row
    col = (j % per_row) * 16
    ref[r, pl.ds(col, 16)] = vec
    return carry

  lax.fori_loop(0, nrows * per_row, body, 0)


def _sc_scatter_body(width, table_hbm, srci_hbm, dsti_hbm, out_hbm,
                     srci_v, dsti_v, rows, table_sh, acc_sh, sems_g, sems_s):
  c = lax.axis_index("c")
  s = lax.axis_index("s")
  t = c * NS + s

  _fill_rows(rows.at[0], K, width, 0.0)
  for j in range(RPT // K):
    pltpu.sync_copy(rows.at[0], acc_sh.at[pl.ds(s * RPT + j * K, K)])

  pltpu.sync_copy(srci_hbm.at[t], srci_v)
  pltpu.sync_copy(dsti_hbm.at[t], dsti_v)
  pltpu.sync_copy(table_hbm.at[pl.ds(s * RPT, RPT)], table_sh.at[pl.ds(s * RPT, RPT)])
  plsc.subcore_barrier()

  def gather(buf, j):
    pltpu.async_copy(table_sh.at[srci_v.at[j]], rows.at[buf], sems_g.at[buf])

  def wait_gather(buf, j):
    pltpu.make_async_copy(table_sh.at[srci_v.at[j]], rows.at[buf],
                          sems_g.at[buf]).wait()

  def scatter(buf, j):
    pltpu.async_copy(rows.at[buf], acc_sh.at[dsti_v.at[j]], sems_s.at[buf],
                     add=True)

  def wait_scatter(buf, j):
    pltpu.make_async_copy(rows.at[buf], acc_sh.at[dsti_v.at[j]],
                          sems_s.at[buf]).wait()

  for b in range(NB):
    gather(b, b)

  def round_(g, carry):
    for half in range(2):
      for b in range(WAVE):
        buf = half * WAVE + b
        j = (2 * g + half) * WAVE + b
        wait_gather(buf, j)
        scatter(buf, j)

      @pl.when(g + 1 < C // NB)
      def _():
        for b in range(WAVE):
          buf = half * WAVE + b
          j = (2 * g + half) * WAVE + b
          wait_scatter(buf, j)
          gather(buf, j + NB)
    return carry

  lax.fori_loop(0, C // NB, round_, 0)
  for buf in range(NB):
    wait_scatter(buf, C - NB + buf)
  plsc.subcore_barrier()

  pltpu.sync_copy(acc_sh.at[pl.ds(s * RPT, RPT)], out_hbm.at[c, pl.ds(s * RPT, RPT)])


def _make_sc_scatter(width):
  mesh = plsc.VectorSubcoreMesh(core_axis_name="c", subcore_axis_name="s")
  return pl.kernel(
      functools.partial(_sc_scatter_body, width),
      out_type=jax.ShapeDtypeStruct((NC, NP, width), jnp.float32),
      mesh=mesh,
      scratch_types=[
          pltpu.VMEM((C, K), jnp.int32),
          pltpu.VMEM((C, K), jnp.int32),
          pltpu.VMEM((NB, K, width), jnp.float32),
          pltpu.VMEM_SHARED((NP, width), jnp.float32),
          pltpu.VMEM_SHARED((NP, width), jnp.float32),
          pltpu.SemaphoreType.DMA((NB,)),
          pltpu.SemaphoreType.DMA((NB,)),
      ],
      compiler_params=pltpu.CompilerParams(use_tc_tiling_on_sc=False),
      name=f"sc_edge_scatter_{width}",
  )


def _sc_deg_body(dsti_hbm, out_hbm, dsti_v, ones_v, zbuf_v, acc_sh):
  c = lax.axis_index("c")
  s = lax.axis_index("s")
  t = c * NS + s

  _fill_rows(zbuf_v, K, 16, 0.0)
  _fill_rows(ones_v, K, 16, 1.0)
  for j in range(RPT // K):
    pltpu.sync_copy(zbuf_v, acc_sh.at[pl.ds(s * RPT + j * K, K)])
  pltpu.sync_copy(dsti_hbm.at[t], dsti_v)
  plsc.subcore_barrier()

  def chunk(j, carry):
    pltpu.sync_copy(ones_v, acc_sh.at[dsti_v.at[j]], add=True)
    return carry

  lax.fori_loop(0, C, chunk, 0)
  plsc.subcore_barrier()
  pltpu.sync_copy(acc_sh.at[pl.ds(s * RPT, RPT)], out_hbm.at[c, pl.ds(s * RPT, RPT)])


def _make_sc_deg():
  mesh = plsc.VectorSubcoreMesh(core_axis_name="c", subcore_axis_name="s")
  return pl.kernel(
      _sc_deg_body,
      out_type=jax.ShapeDtypeStruct((NC, NP, 16), jnp.float32),
      mesh=mesh,
      scratch_types=[
          pltpu.VMEM((C, K), jnp.int32),
          pltpu.VMEM((K, 16), jnp.float32),
          pltpu.VMEM((K, 16), jnp.float32),
          pltpu.VMEM_SHARED((NP, 16), jnp.float32),
      ],
      compiler_params=pltpu.CompilerParams(use_tc_tiling_on_sc=False),
      name="sc_degree",
  )


_PREC = lax.Precision.HIGHEST


def _bn(h, g, b, m, v):
  return g * (h - m) / jnp.sqrt(v + 1e-3) + b


def _mm(a, w):
  return jnp.dot(a.astype(jnp.bfloat16), w.astype(jnp.bfloat16),
                 preferred_element_type=jnp.float32)


def _tc1_body(x_ref, deg_ref, w_ref, g_ref, b_ref, m_ref, v_ref, y_ref, rs_ref):
  deg = deg_ref[0] + deg_ref[1] + 1.0
  rs = lax.rsqrt(deg)
  rs_ref[...] = rs
  h0 = _bn(x_ref[...], g_ref[...], b_ref[...], m_ref[...], v_ref[...])
  y_ref[...] = rs[:, 0:1] * _mm(h0, w_ref[...])


def _tc1(xp, degp, w1, g1, b1, m1, v1):
  return pl.pallas_call(
      _tc1_body,
      grid=(NSTEPS,),
      in_specs=[
          pl.BlockSpec((BLK, D), lambda b: (b, 0)),
          pl.BlockSpec((NC, BLK, 16), lambda b: (0, b, 0)),
          pl.BlockSpec((D, H), lambda b: (0, 0)),
          pl.BlockSpec((1, D), lambda b: (0, 0)),
          pl.BlockSpec((1, D), lambda b: (0, 0)),
          pl.BlockSpec((1, D), lambda b: (0, 0)),
          pl.BlockSpec((1, D), lambda b: (0, 0)),
      ],
      out_specs=[
          pl.BlockSpec((BLK, H), lambda b: (b, 0)),
          pl.BlockSpec((BLK, 16), lambda b: (b, 0)),
      ],
      out_shape=[
          jax.ShapeDtypeStruct((NP, H), jnp.float32),
          jax.ShapeDtypeStruct((NP, 16), jnp.float32),
      ],
  )(xp, degp, w1, g1, b1, m1, v1)


def _tc2_body(y1_ref, s_ref, rs_ref, w_ref, bc_ref, g_ref, b_ref, m_ref, v_ref,
              y2_ref):
  r = rs_ref[...][:, 0:1]
  a = r * (y1_ref[...] + s_ref[0] + s_ref[1]) + bc_ref[...]
  h = jnp.maximum(a, 0.0)
  hb = _bn(h, g_ref[...], b_ref[...], m_ref[...], v_ref[...])
  y2_ref[...] = r * _mm(hb, w_ref[...])


def _tc2(y1, sp, rs16, w2, bconv1, g2, b2, m2, v2):
  return pl.pallas_call(
      _tc2_body,
      grid=(NSTEPS,),
      in_specs=[
          pl.BlockSpec((BLK, H), lambda b: (b, 0)),
          pl.BlockSpec((NC, BLK, H), lambda b: (0, b, 0)),
          pl.BlockSpec((BLK, 16), lambda b: (b, 0)),
          pl.BlockSpec((H, H), lambda b: (0, 0)),
          pl.BlockSpec((1, H), lambda b: (0, 0)),
          pl.BlockSpec((1, H), lambda b: (0, 0)),
          pl.BlockSpec((1, H), lambda b: (0, 0)),
          pl.BlockSpec((1, H), lambda b: (0, 0)),
          pl.BlockSpec((1, H), lambda b: (0, 0)),
      ],
      out_specs=pl.BlockSpec((BLK, H), lambda b: (b, 0)),
      out_shape=jax.ShapeDtypeStruct((NP, H), jnp.float32),
  )(y1, sp, rs16, w2, bconv1, g2, b2, m2, v2)


def _tc3_body(y2_ref, s_ref, rs_ref, wfa_ref, cfa_ref, bc_ref, g_ref, b_ref,
              m_ref, v_ref, ip_ref,
              wd1_ref, bd1_ref, wd2_ref, bd2_ref, out_ref, pool_acc):
  step = pl.program_id(0)
  r = rs_ref[...][:, 0:1]
  a = r * (y2_ref[...] + s_ref[0] + s_ref[1]) + bc_ref[...]
  h = jnp.maximum(a, 0.0)
  hb = _bn(h, g_ref[...], b_ref[...], m_ref[...], v_ref[...])
  z = _mm(hb, wfa_ref[...]) + cfa_ref[...]
  p = z[:, :P] * jax.nn.sigmoid(z[:, P:])
  ids = ip_ref[...]
  onehot = (ids == lax.broadcasted_iota(jnp.int32, (G, BLK), 0)).astype(jnp.float32)
  part = jnp.dot(onehot, p, preferred_element_type=jnp.float32, precision=_PREC)

  @pl.when(step == 0)
  def _():
    pool_acc[...] = part

  @pl.when(step > 0)
  def _():
    pool_acc[...] += part

  @pl.when(step == pl.num_programs(0) - 1)
  def _():
    d1 = jnp.maximum(_mm(pool_acc[...], wd1_ref[...]) + bd1_ref[...], 0.0)
    out_ref[...] = _mm(d1, wd2_ref[...]) + bd2_ref[...]


def _tc3(y2, sp, rs16, wfa, cfa, bconv2, g3, b3, m3, v3, ip, wd1, bd1, wd2, bd2):
  return pl.pallas_call(
      _tc3_body,
      grid=(NSTEPS,),
      in_specs=[
          pl.BlockSpec((BLK, H), lambda b: (b, 0)),
          pl.BlockSpec((NC, BLK, H), lambda b: (0, b, 0)),
          pl.BlockSpec((BLK, 16), lambda b: (b, 0)),
          pl.BlockSpec((H, 2 * P), lambda b: (0, 0)),
          pl.BlockSpec((1, 2 * P), lambda b: (0, 0)),
          pl.BlockSpec((1, H), lambda b: (0, 0)),
          pl.BlockSpec((1, H), lambda b: (0, 0)),
          pl.BlockSpec((1, H), lambda b: (0, 0)),
          pl.BlockSpec((1, H), lambda b: (0, 0)),
          pl.BlockSpec((1, H), lambda b: (0, 0)),
          pl.BlockSpec((1, BLK), lambda b: (0, b)),
          pl.BlockSpec((P, H), lambda b: (0, 0)),
          pl.BlockSpec((1, H), lambda b: (0, 0)),
          pl.BlockSpec((H, 1), lambda b: (0, 0)),
          pl.BlockSpec((1, 1), lambda b: (0, 0)),
      ],
      out_specs=pl.BlockSpec((G, 1), lambda b: (0, 0)),
      out_shape=jax.ShapeDtypeStruct((G, 1), jnp.float32),
      scratch_shapes=[pltpu.VMEM((G, P), jnp.float32)],
  )(y2, sp, rs16, wfa, cfa, bconv2, g3, b3, m3, v3, ip, wd1, bd1, wd2, bd2)


def kernel(x, edge_index, i, gamma1, beta1, mean1, var1, W1, b1,
           gamma2, beta2, mean2, var2, W2, b2,
           gamma3, beta3, mean3, var3,
           Wf, bf, Wa, ba, Wd1, bd1, Wd2, bd2):
  f32 = jnp.float32
  wfa = jnp.concatenate([Wf, Wa], axis=1)
  cfa = jnp.concatenate([bf, ba])[None, :]

  src = edge_index[0].astype(jnp.int32).reshape(NTILES, EPT)
  dst = edge_index[1].astype(jnp.int32).reshape(NTILES, EPT)
  srci = jnp.pad(src, ((0, 0), (0, EPT_PAD - EPT))).reshape(NTILES, C, K)
  dsti = jnp.pad(dst, ((0, 0), (0, EPT_PAD - EPT)),
                 constant_values=N).reshape(NTILES, C, K)

  xp = jnp.pad(x.astype(f32), ((0, NP - N), (0, 0)))
  ip = jnp.pad(i.astype(jnp.int32), (0, NP - N), constant_values=G).reshape(1, NP)

  degp = _make_sc_deg()(dsti)
  y1, rs16 = _tc1(xp, degp, W1, gamma1[None, :], beta1[None, :],
                  mean1[None, :], var1[None, :])
  s1p = _make_sc_scatter(H)(y1, srci, dsti)
  y2 = _tc2(y1, s1p, rs16, W2, b1[None, :], gamma2[None, :], beta2[None, :],
            mean2[None, :], var2[None, :])
  s2p = _make_sc_scatter(H)(y2, srci, dsti)
  return _tc3(y2, s2p, rs16, wfa, cfa, b2[None, :], gamma3[None, :],
              beta3[None, :], mean3[None, :], var3[None, :], ip,
              Wd1, bd1[None, :], Wd2, bd2[None, :])


# speedup vs baseline: 1.0251x; 1.0251x over previous
import functools

import jax
import jax.numpy as jnp
from jax import lax
from jax.experimental import pallas as pl
from jax.experimental.pallas import tpu as pltpu
from jax.experimental.pallas import tpu_sc as plsc


N = 10000
E = 320000
D = 128
H = 64
P = 32
G = 128

NP = 10240
BLK = 1024
NSTEPS = NP // BLK

NC = 2
NS = 16
NTILES = NC * NS
EPT = E // NTILES
K = 64
NB = 4
WAVE = NB // 2
C = 160
EPT_PAD = C * K
RPT = NP // NS


def _fill_rows(ref, nrows, ncols, value):
  vec = jnp.full((16,), value, jnp.float32)
  per_row = ncols // 16

  def body(j, carry):
    r = j // per_row
    col = (j % per_row) * 16
    ref[r, pl.ds(col, 16)] = vec
    return carry

  lax.fori_loop(0, nrows * per_row, body, 0)


def _sc_scatter_body(width, table_hbm, srci_hbm, dsti_hbm, out_hbm,
                     srci_v, dsti_v, rows, table_sh, acc_sh, sems_g, sems_s):
  c = lax.axis_index("c")
  s = lax.axis_index("s")
  t = c * NS + s

  _fill_rows(rows.at[0], K, width, 0.0)
  for j in range(RPT // K):
    pltpu.sync_copy(rows.at[0], acc_sh.at[pl.ds(s * RPT + j * K, K)])

  pltpu.sync_copy(srci_hbm.at[t], srci_v)
  pltpu.sync_copy(dsti_hbm.at[t], dsti_v)
  pltpu.sync_copy(table_hbm.at[pl.ds(s * RPT, RPT)], table_sh.at[pl.ds(s * RPT, RPT)])
  plsc.subcore_barrier()

  def gather(buf, j):
    pltpu.async_copy(table_sh.at[srci_v.at[j]], rows.at[buf], sems_g.at[buf])

  def wait_gather(buf, j):
    pltpu.make_async_copy(table_sh.at[srci_v.at[j]], rows.at[buf],
                          sems_g.at[buf]).wait()

  def scatter(buf, j):
    pltpu.async_copy(rows.at[buf], acc_sh.at[dsti_v.at[j]], sems_s.at[buf],
                     add=True)

  def wait_scatter(buf, j):
    pltpu.make_async_copy(rows.at[buf], acc_sh.at[dsti_v.at[j]],
                          sems_s.at[buf]).wait()

  for b in range(NB):
    gather(b, b)

  def round_(g, carry):
    for half in range(2):
      for b in range(WAVE):
        buf = half * WAVE + b
        j = (2 * g + half) * WAVE + b
        wait_gather(buf, j)
        scatter(buf, j)

      @pl.when(g + 1 < C // NB)
      def _():
        for b in range(WAVE):
          buf = half * WAVE + b
          j = (2 * g + half) * WAVE + b
          wait_scatter(buf, j)
          gather(buf, j + NB)
    return carry

  lax.fori_loop(0, C // NB, round_, 0)
  for buf in range(NB):
    wait_scatter(buf, C - NB + buf)
  plsc.subcore_barrier()

  pltpu.sync_copy(acc_sh.at[pl.ds(s * RPT, RPT)], out_hbm.at[c, pl.ds(s * RPT, RPT)])


def _make_sc_scatter(width):
  mesh = plsc.VectorSubcoreMesh(core_axis_name="c", subcore_axis_name="s")
  return pl.kernel(
      functools.partial(_sc_scatter_body, width),
      out_type=jax.ShapeDtypeStruct((NC, NP, width), jnp.float32),
      mesh=mesh,
      scratch_types=[
          pltpu.VMEM((C, K), jnp.int32),
          pltpu.VMEM((C, K), jnp.int32),
          pltpu.VMEM((NB, K, width), jnp.float32),
          pltpu.VMEM_SHARED((NP, width), jnp.float32),
          pltpu.VMEM_SHARED((NP, width), jnp.float32),
          pltpu.SemaphoreType.DMA((NB,)),
          pltpu.SemaphoreType.DMA((NB,)),
      ],
      compiler_params=pltpu.CompilerParams(use_tc_tiling_on_sc=False),
      name=f"sc_edge_scatter_{width}",
  )


DK = 128
DC = EPT_PAD // DK


def _sc_deg_body(dsti_hbm, out_hbm, dsti_v, ones_v, zbuf_v, acc_sh):
  c = lax.axis_index("c")
  s = lax.axis_index("s")
  t = c * NS + s

  _fill_rows(zbuf_v, DK, 16, 0.0)
  _fill_rows(ones_v, DK, 16, 1.0)
  for j in range(RPT // DK):
    pltpu.sync_copy(zbuf_v, acc_sh.at[pl.ds(s * RPT + j * DK, DK)])
  pltpu.sync_copy(dsti_hbm.at[t], dsti_v)
  plsc.subcore_barrier()

  def chunk(j, carry):
    pltpu.sync_copy(ones_v, acc_sh.at[dsti_v.at[j]], add=True)
    return carry

  lax.fori_loop(0, DC, chunk, 0)
  plsc.subcore_barrier()
  pltpu.sync_copy(acc_sh.at[pl.ds(s * RPT, RPT)], out_hbm.at[c, pl.ds(s * RPT, RPT)])


def _make_sc_deg():
  mesh = plsc.VectorSubcoreMesh(core_axis_name="c", subcore_axis_name="s")
  return pl.kernel(
      _sc_deg_body,
      out_type=jax.ShapeDtypeStruct((NC, NP, 16), jnp.float32),
      mesh=mesh,
      scratch_types=[
          pltpu.VMEM((DC, DK), jnp.int32),
          pltpu.VMEM((DK, 16), jnp.float32),
          pltpu.VMEM((DK, 16), jnp.float32),
          pltpu.VMEM_SHARED((NP, 16), jnp.float32),
      ],
      compiler_params=pltpu.CompilerParams(use_tc_tiling_on_sc=False),
      name="sc_degree",
  )


_PREC = lax.Precision.HIGHEST


def _bn(h, g, b, m, v):
  return g * (h - m) / jnp.sqrt(v + 1e-3) + b


def _mm(a, w):
  return jnp.dot(a.astype(jnp.bfloat16), w.astype(jnp.bfloat16),
                 preferred_element_type=jnp.float32)


def _tc1_body(x_ref, deg_ref, w_ref, g_ref, b_ref, m_ref, v_ref, y_ref, rs_ref):
  deg = deg_ref[0] + deg_ref[1] + 1.0
  rs = lax.rsqrt(deg)
  rs_ref[...] = rs
  h0 = _bn(x_ref[...], g_ref[...], b_ref[...], m_ref[...], v_ref[...])
  y_ref[...] = rs[:, 0:1] * _mm(h0, w_ref[...])


def _tc1(xp, degp, w1, g1, b1, m1, v1):
  return pl.pallas_call(
      _tc1_body,
      grid=(NSTEPS,),
      in_specs=[
          pl.BlockSpec((BLK, D), lambda b: (b, 0)),
          pl.BlockSpec((NC, BLK, 16), lambda b: (0, b, 0)),
          pl.BlockSpec((D, H), lambda b: (0, 0)),
          pl.BlockSpec((1, D), lambda b: (0, 0)),
          pl.BlockSpec((1, D), lambda b: (0, 0)),
          pl.BlockSpec((1, D), lambda b: (0, 0)),
          pl.BlockSpec((1, D), lambda b: (0, 0)),
      ],
      out_specs=[
          pl.BlockSpec((BLK, H), lambda b: (b, 0)),
          pl.BlockSpec((BLK, 16), lambda b: (b, 0)),
      ],
      out_shape=[
          jax.ShapeDtypeStruct((NP, H), jnp.float32),
          jax.ShapeDtypeStruct((NP, 16), jnp.float32),
      ],
  )(xp, degp, w1, g1, b1, m1, v1)


def _tc2_body(y1_ref, s_ref, rs_ref, w_ref, bc_ref, g_ref, b_ref, m_ref, v_ref,
              y2_ref):
  r = rs_ref[...][:, 0:1]
  a = r * (y1_ref[...] + s_ref[0] + s_ref[1]) + bc_ref[...]
  h = jnp.maximum(a, 0.0)
  hb = _bn(h, g_ref[...], b_ref[...], m_ref[...], v_ref[...])
  y2_ref[...] = r * _mm(hb, w_ref[...])


def _tc2(y1, sp, rs16, w2, bconv1, g2, b2, m2, v2):
  return pl.pallas_call(
      _tc2_body,
      grid=(NSTEPS,),
      in_specs=[
          pl.BlockSpec((BLK, H), lambda b: (b, 0)),
          pl.BlockSpec((NC, BLK, H), lambda b: (0, b, 0)),
          pl.BlockSpec((BLK, 16), lambda b: (b, 0)),
          pl.BlockSpec((H, H), lambda b: (0, 0)),
          pl.BlockSpec((1, H), lambda b: (0, 0)),
          pl.BlockSpec((1, H), lambda b: (0, 0)),
          pl.BlockSpec((1, H), lambda b: (0, 0)),
          pl.BlockSpec((1, H), lambda b: (0, 0)),
          pl.BlockSpec((1, H), lambda b: (0, 0)),
      ],
      out_specs=pl.BlockSpec((BLK, H), lambda b: (b, 0)),
      out_shape=jax.ShapeDtypeStruct((NP, H), jnp.float32),
  )(y1, sp, rs16, w2, bconv1, g2, b2, m2, v2)


def _tc3_body(y2_ref, s_ref, rs_ref, wfa_ref, cfa_ref, bc_ref, g_ref, b_ref,
              m_ref, v_ref, ip_ref,
              wd1_ref, bd1_ref, wd2_ref, bd2_ref, out_ref, pool_acc):
  step = pl.program_id(0)
  r = rs_ref[...][:, 0:1]
  a = r * (y2_ref[...] + s_ref[0] + s_ref[1]) + bc_ref[...]
  h = jnp.maximum(a, 0.0)
  hb = _bn(h, g_ref[...], b_ref[...], m_ref[...], v_ref[...])
  z = _mm(hb, wfa_ref[...]) + cfa_ref[...]
  p = z[:, :P] * jax.nn.sigmoid(z[:, P:])
  ids = ip_ref[...]
  onehot = (ids == lax.broadcasted_iota(jnp.int32, (G, BLK), 0)).astype(jnp.float32)
  part = jnp.dot(onehot, p, preferred_element_type=jnp.float32, precision=_PREC)

  @pl.when(step == 0)
  def _():
    pool_acc[...] = part

  @pl.when(step > 0)
  def _():
    pool_acc[...] += part

  @pl.when(step == pl.num_programs(0) - 1)
  def _():
    d1 = jnp.maximum(_mm(pool_acc[...], wd1_ref[...]) + bd1_ref[...], 0.0)
    out_ref[...] = _mm(d1, wd2_ref[...]) + bd2_ref[...]


def _tc3(y2, sp, rs16, wfa, cfa, bconv2, g3, b3, m3, v3, ip, wd1, bd1, wd2, bd2):
  return pl.pallas_call(
      _tc3_body,
      grid=(NSTEPS,),
      in_specs=[
          pl.BlockSpec((BLK, H), lambda b: (b, 0)),
          pl.BlockSpec((NC, BLK, H), lambda b: (0, b, 0)),
          pl.BlockSpec((BLK, 16), lambda b: (b, 0)),
          pl.BlockSpec((H, 2 * P), lambda b: (0, 0)),
          pl.BlockSpec((1, 2 * P), lambda b: (0, 0)),
          pl.BlockSpec((1, H), lambda b: (0, 0)),
          pl.BlockSpec((1, H), lambda b: (0, 0)),
          pl.BlockSpec((1, H), lambda b: (0, 0)),
          pl.BlockSpec((1, H), lambda b: (0, 0)),
          pl.BlockSpec((1, H), lambda b: (0, 0)),
          pl.BlockSpec((1, BLK), lambda b: (0, b)),
          pl.BlockSpec((P, H), lambda b: (0, 0)),
          pl.BlockSpec((1, H), lambda b: (0, 0)),
          pl.BlockSpec((H, 1), lambda b: (0, 0)),
          pl.BlockSpec((1, 1), lambda b: (0, 0)),
      ],
      out_specs=pl.BlockSpec((G, 1), lambda b: (0, 0)),
      out_shape=jax.ShapeDtypeStruct((G, 1), jnp.float32),
      scratch_shapes=[pltpu.VMEM((G, P), jnp.float32)],
  )(y2, sp, rs16, wfa, cfa, bconv2, g3, b3, m3, v3, ip, wd1, bd1, wd2, bd2)


def kernel(x, edge_index, i, gamma1, beta1, mean1, var1, W1, b1,
           gamma2, beta2, mean2, var2, W2, b2,
           gamma3, beta3, mean3, var3,
           Wf, bf, Wa, ba, Wd1, bd1, Wd2, bd2):
  f32 = jnp.float32
  wfa = jnp.concatenate([Wf, Wa], axis=1)
  cfa = jnp.concatenate([bf, ba])[None, :]

  src = edge_index[0].astype(jnp.int32).reshape(NTILES, EPT)
  dst = edge_index[1].astype(jnp.int32).reshape(NTILES, EPT)
  srci = jnp.pad(src, ((0, 0), (0, EPT_PAD - EPT))).reshape(NTILES, C, K)
  dsti = jnp.pad(dst, ((0, 0), (0, EPT_PAD - EPT)),
                 constant_values=N).reshape(NTILES, C, K)

  xp = jnp.pad(x.astype(f32), ((0, NP - N), (0, 0)))
  ip = jnp.pad(i.astype(jnp.int32), (0, NP - N), constant_values=G).reshape(1, NP)

  degp = _make_sc_deg()(dsti.reshape(NTILES, DC, DK))
  y1, rs16 = _tc1(xp, degp, W1, gamma1[None, :], beta1[None, :],
                  mean1[None, :], var1[None, :])
  s1p = _make_sc_scatter(H)(y1, srci, dsti)
  y2 = _tc2(y1, s1p, rs16, W2, b1[None, :], gamma2[None, :], beta2[None, :],
            mean2[None, :], var2[None, :])
  s2p = _make_sc_scatter(H)(y2, srci, dsti)
  return _tc3(y2, s2p, rs16, wfa, cfa, b2[None, :], gamma3[None, :],
              beta3[None, :], mean3[None, :], var3[None, :], ip,
              Wd1, bd1[None, :], Wd2, bd2[None, :])


# speedup vs baseline: 1.0583x; 1.0324x over previous
import functools

import jax
import jax.numpy as jnp
from jax import lax
from jax.experimental import pallas as pl
from jax.experimental.pallas import tpu as pltpu
from jax.experimental.pallas import tpu_sc as plsc


N = 10000
E = 320000
D = 128
H = 64
P = 32
G = 128

NP = 10240
BLK = 1024
NSTEPS = NP // BLK

NC = 2
NS = 16
NTILES = NC * NS
EPT = E // NTILES
K = 64
NB = 4
WAVE = NB // 2
C = 160
EPT_PAD = C * K
RPT = NP // NS


def _fill_rows(ref, nrows, ncols, value):
  vec = jnp.full((16,), value, jnp.float32)
  per_row = ncols // 16

  def body(j, carry):
    r = j // per_row
    col = (j % per_row) * 16
    ref[r, pl.ds(col, 16)] = vec
    return carry

  lax.fori_loop(0, nrows * per_row, body, 0)


def _sc_scatter_body(width, table_hbm, srci_hbm, dsti_hbm, out_hbm,
                     srci_v, dsti_v, rows, table_sh, acc_sh, sems_g, sems_s):
  c = lax.axis_index("c")
  s = lax.axis_index("s")
  t = c * NS + s

  pltpu.async_copy(srci_hbm.at[t], srci_v, sems_g.at[0])
  pltpu.async_copy(dsti_hbm.at[t], dsti_v, sems_g.at[1])
  pltpu.async_copy(table_hbm.at[pl.ds(s * RPT, RPT)],
                   table_sh.at[pl.ds(s * RPT, RPT)], sems_g.at[2])
  _fill_rows(rows.at[0], K, width, 0.0)
  for j in range(RPT // K):
    pltpu.sync_copy(rows.at[0], acc_sh.at[pl.ds(s * RPT + j * K, K)])
  pltpu.make_async_copy(srci_hbm.at[t], srci_v, sems_g.at[0]).wait()
  pltpu.make_async_copy(dsti_hbm.at[t], dsti_v, sems_g.at[1]).wait()
  pltpu.make_async_copy(table_hbm.at[pl.ds(s * RPT, RPT)],
                        table_sh.at[pl.ds(s * RPT, RPT)], sems_g.at[2]).wait()
  plsc.subcore_barrier()

  def gather(buf, j):
    pltpu.async_copy(table_sh.at[srci_v.at[j]], rows.at[buf], sems_g.at[buf])

  def wait_gather(buf, j):
    pltpu.make_async_copy(table_sh.at[srci_v.at[j]], rows.at[buf],
                          sems_g.at[buf]).wait()

  def scatter(buf, j):
    pltpu.async_copy(rows.at[buf], acc_sh.at[dsti_v.at[j]], sems_s.at[buf],
                     add=True)

  def wait_scatter(buf, j):
    pltpu.make_async_copy(rows.at[buf], acc_sh.at[dsti_v.at[j]],
                          sems_s.at[buf]).wait()

  for b in range(NB):
    gather(b, b)

  def round_(g, carry):
    for half in range(2):
      for b in range(WAVE):
        buf = half * WAVE + b
        j = (2 * g + half) * WAVE + b
        wait_gather(buf, j)
        scatter(buf, j)

      @pl.when(g + 1 < C // NB)
      def _():
        for b in range(WAVE):
          buf = half * WAVE + b
          j = (2 * g + half) * WAVE + b
          wait_scatter(buf, j)
          gather(buf, j + NB)
    return carry

  lax.fori_loop(0, C // NB, round_, 0)
  for buf in range(NB):
    wait_scatter(buf, C - NB + buf)
  plsc.subcore_barrier()

  pltpu.sync_copy(acc_sh.at[pl.ds(s * RPT, RPT)], out_hbm.at[c, pl.ds(s * RPT, RPT)])


def _make_sc_scatter(width):
  mesh = plsc.VectorSubcoreMesh(core_axis_name="c", subcore_axis_name="s")
  return pl.kernel(
      functools.partial(_sc_scatter_body, width),
      out_type=jax.ShapeDtypeStruct((NC, NP, width), jnp.float32),
      mesh=mesh,
      scratch_types=[
          pltpu.VMEM((C, K), jnp.int32),
          pltpu.VMEM((C, K), jnp.int32),
          pltpu.VMEM((NB, K, width), jnp.float32),
          pltpu.VMEM_SHARED((NP, width), jnp.float32),
          pltpu.VMEM_SHARED((NP, width), jnp.float32),
          pltpu.SemaphoreType.DMA((NB,)),
          pltpu.SemaphoreType.DMA((NB,)),
      ],
      compiler_params=pltpu.CompilerParams(use_tc_tiling_on_sc=False),
      name=f"sc_edge_scatter_{width}",
  )


DK = 128
DC = EPT_PAD // DK


def _sc_deg_body(dsti_hbm, out_hbm, dsti_v, ones_v, zbuf_v, acc_sh):
  c = lax.axis_index("c")
  s = lax.axis_index("s")
  t = c * NS + s

  _fill_rows(zbuf_v, DK, 16, 0.0)
  _fill_rows(ones_v, DK, 16, 1.0)
  for j in range(RPT // DK):
    pltpu.sync_copy(zbuf_v, acc_sh.at[pl.ds(s * RPT + j * DK, DK)])
  pltpu.sync_copy(dsti_hbm.at[t], dsti_v)
  plsc.subcore_barrier()

  def chunk(j, carry):
    pltpu.sync_copy(ones_v, acc_sh.at[dsti_v.at[j]], add=True)
    return carry

  lax.fori_loop(0, DC, chunk, 0)
  plsc.subcore_barrier()
  pltpu.sync_copy(acc_sh.at[pl.ds(s * RPT, RPT)], out_hbm.at[c, pl.ds(s * RPT, RPT)])


def _make_sc_deg():
  mesh = plsc.VectorSubcoreMesh(core_axis_name="c", subcore_axis_name="s")
  return pl.kernel(
      _sc_deg_body,
      out_type=jax.ShapeDtypeStruct((NC, NP, 16), jnp.float32),
      mesh=mesh,
      scratch_types=[
          pltpu.VMEM((DC, DK), jnp.int32),
          pltpu.VMEM((DK, 16), jnp.float32),
          pltpu.VMEM((DK, 16), jnp.float32),
          pltpu.VMEM_SHARED((NP, 16), jnp.float32),
      ],
      compiler_params=pltpu.CompilerParams(use_tc_tiling_on_sc=False),
      name="sc_degree",
  )


_PREC = lax.Precision.HIGHEST


def _bn(h, g, b, m, v):
  return g * (h - m) / jnp.sqrt(v + 1e-3) + b


def _mm(a, w):
  return jnp.dot(a.astype(jnp.bfloat16), w.astype(jnp.bfloat16),
                 preferred_element_type=jnp.float32)


def _tc1_body(x_ref, deg_ref, w_ref, g_ref, b_ref, m_ref, v_ref, y_ref, rs_ref):
  deg = deg_ref[0] + deg_ref[1] + 1.0
  rs = lax.rsqrt(deg)
  rs_ref[...] = rs
  h0 = _bn(x_ref[...], g_ref[...], b_ref[...], m_ref[...], v_ref[...])
  y_ref[...] = rs[:, 0:1] * _mm(h0, w_ref[...])


def _tc1(xp, degp, w1, g1, b1, m1, v1):
  return pl.pallas_call(
      _tc1_body,
      grid=(NSTEPS,),
      in_specs=[
          pl.BlockSpec((BLK, D), lambda b: (b, 0)),
          pl.BlockSpec((NC, BLK, 16), lambda b: (0, b, 0)),
          pl.BlockSpec((D, H), lambda b: (0, 0)),
          pl.BlockSpec((1, D), lambda b: (0, 0)),
          pl.BlockSpec((1, D), lambda b: (0, 0)),
          pl.BlockSpec((1, D), lambda b: (0, 0)),
          pl.BlockSpec((1, D), lambda b: (0, 0)),
      ],
      out_specs=[
          pl.BlockSpec((BLK, H), lambda b: (b, 0)),
          pl.BlockSpec((BLK, 16), lambda b: (b, 0)),
      ],
      out_shape=[
          jax.ShapeDtypeStruct((NP, H), jnp.float32),
          jax.ShapeDtypeStruct((NP, 16), jnp.float32),
      ],
  )(xp, degp, w1, g1, b1, m1, v1)


def _tc2_body(y1_ref, s_ref, rs_ref, w_ref, bc_ref, g_ref, b_ref, m_ref, v_ref,
              y2_ref):
  r = rs_ref[...][:, 0:1]
  a = r * (y1_ref[...] + s_ref[0] + s_ref[1]) + bc_ref[...]
  h = jnp.maximum(a, 0.0)
  hb = _bn(h, g_ref[...], b_ref[...], m_ref[...], v_ref[...])
  y2_ref[...] = r * _mm(hb, w_ref[...])


def _tc2(y1, sp, rs16, w2, bconv1, g2, b2, m2, v2):
  return pl.pallas_call(
      _tc2_body,
      grid=(NSTEPS,),
      in_specs=[
          pl.BlockSpec((BLK, H), lambda b: (b, 0)),
          pl.BlockSpec((NC, BLK, H), lambda b: (0, b, 0)),
          pl.BlockSpec((BLK, 16), lambda b: (b, 0)),
          pl.BlockSpec((H, H), lambda b: (0, 0)),
          pl.BlockSpec((1, H), lambda b: (0, 0)),
          pl.BlockSpec((1, H), lambda b: (0, 0)),
          pl.BlockSpec((1, H), lambda b: (0, 0)),
          pl.BlockSpec((1, H), lambda b: (0, 0)),
          pl.BlockSpec((1, H), lambda b: (0, 0)),
      ],
      out_specs=pl.BlockSpec((BLK, H), lambda b: (b, 0)),
      out_shape=jax.ShapeDtypeStruct((NP, H), jnp.float32),
  )(y1, sp, rs16, w2, bconv1, g2, b2, m2, v2)


def _tc3_body(y2_ref, s_ref, rs_ref, wfa_ref, cfa_ref, bc_ref, g_ref, b_ref,
              m_ref, v_ref, ip_ref,
              wd1_ref, bd1_ref, wd2_ref, bd2_ref, out_ref, pool_acc):
  step = pl.program_id(0)
  r = rs_ref[...][:, 0:1]
  a = r * (y2_ref[...] + s_ref[0] + s_ref[1]) + bc_ref[...]
  h = jnp.maximum(a, 0.0)
  hb = _bn(h, g_ref[...], b_ref[...], m_ref[...], v_ref[...])
  z = _mm(hb, wfa_ref[...]) + cfa_ref[...]
  p = z[:, :P] * jax.nn.sigmoid(z[:, P:])
  ids = ip_ref[...]
  onehot = (ids == lax.broadcasted_iota(jnp.int32, (G, BLK), 0)).astype(jnp.float32)
  part = jnp.dot(onehot, p, preferred_element_type=jnp.float32, precision=_PREC)

  @pl.when(step == 0)
  def _():
    pool_acc[...] = part

  @pl.when(step > 0)
  def _():
    pool_acc[...] += part

  @pl.when(step == pl.num_programs(0) - 1)
  def _():
    d1 = jnp.maximum(_mm(pool_acc[...], wd1_ref[...]) + bd1_ref[...], 0.0)
    out_ref[...] = _mm(d1, wd2_ref[...]) + bd2_ref[...]


def _tc3(y2, sp, rs16, wfa, cfa, bconv2, g3, b3, m3, v3, ip, wd1, bd1, wd2, bd2):
  return pl.pallas_call(
      _tc3_body,
      grid=(NSTEPS,),
      in_specs=[
          pl.BlockSpec((BLK, H), lambda b: (b, 0)),
          pl.BlockSpec((NC, BLK, H), lambda b: (0, b, 0)),
          pl.BlockSpec((BLK, 16), lambda b: (b, 0)),
          pl.BlockSpec((H, 2 * P), lambda b: (0, 0)),
          pl.BlockSpec((1, 2 * P), lambda b: (0, 0)),
          pl.BlockSpec((1, H), lambda b: (0, 0)),
          pl.BlockSpec((1, H), lambda b: (0, 0)),
          pl.BlockSpec((1, H), lambda b: (0, 0)),
          pl.BlockSpec((1, H), lambda b: (0, 0)),
          pl.BlockSpec((1, H), lambda b: (0, 0)),
          pl.BlockSpec((1, BLK), lambda b: (0, b)),
          pl.BlockSpec((P, H), lambda b: (0, 0)),
          pl.BlockSpec((1, H), lambda b: (0, 0)),
          pl.BlockSpec((H, 1), lambda b: (0, 0)),
          pl.BlockSpec((1, 1), lambda b: (0, 0)),
      ],
      out_specs=pl.BlockSpec((G, 1), lambda b: (0, 0)),
      out_shape=jax.ShapeDtypeStruct((G, 1), jnp.float32),
      scratch_shapes=[pltpu.VMEM((G, P), jnp.float32)],
  )(y2, sp, rs16, wfa, cfa, bconv2, g3, b3, m3, v3, ip, wd1, bd1, wd2, bd2)


def kernel(x, edge_index, i, gamma1, beta1, mean1, var1, W1, b1,
           gamma2, beta2, mean2, var2, W2, b2,
           gamma3, beta3, mean3, var3,
           Wf, bf, Wa, ba, Wd1, bd1, Wd2, bd2):
  f32 = jnp.float32
  wfa = jnp.concatenate([Wf, Wa], axis=1)
  cfa = jnp.concatenate([bf, ba])[None, :]

  src = edge_index[0].astype(jnp.int32).reshape(NTILES, EPT)
  dst = edge_index[1].astype(jnp.int32).reshape(NTILES, EPT)
  srci = jnp.pad(src, ((0, 0), (0, EPT_PAD - EPT))).reshape(NTILES, C, K)
  dsti = jnp.pad(dst, ((0, 0), (0, EPT_PAD - EPT)),
                 constant_values=N).reshape(NTILES, C, K)

  xp = jnp.pad(x.astype(f32), ((0, NP - N), (0, 0)))
  ip = jnp.pad(i.astype(jnp.int32), (0, NP - N), constant_values=G).reshape(1, NP)

  degp = _make_sc_deg()(dsti.reshape(NTILES, DC, DK))
  y1, rs16 = _tc1(xp, degp, W1, gamma1[None, :], beta1[None, :],
                  mean1[None, :], var1[None, :])
  s1p = _make_sc_scatter(H)(y1, srci, dsti)
  y2 = _tc2(y1, s1p, rs16, W2, b1[None, :], gamma2[None, :], beta2[None, :],
            mean2[None, :], var2[None, :])
  s2p = _make_sc_scatter(H)(y2, srci, dsti)
  return _tc3(y2, s2p, rs16, wfa, cfa, b2[None, :], gamma3[None, :],
              beta3[None, :], mean3[None, :], var3[None, :], ip,
              Wd1, bd1[None, :], Wd2, bd2[None, :])


# speedup vs baseline: 1.0666x; 1.0078x over previous
import functools

import jax
import jax.numpy as jnp
from jax import lax
from jax.experimental import pallas as pl
from jax.experimental.pallas import tpu as pltpu
from jax.experimental.pallas import tpu_sc as plsc


N = 10000
E = 320000
D = 128
H = 64
P = 32
G = 128

NP = 10240
BLK = 1024
NSTEPS = NP // BLK

NC = 2
NS = 16
NTILES = NC * NS
EPT = E // NTILES
K = 64
NB = 4
WAVE = NB // 2
C = 160
EPT_PAD = C * K
RPT = NP // NS


def _fill_rows(ref, nrows, ncols, value):
  vec = jnp.full((16,), value, jnp.float32)
  per_row = ncols // 16

  def body(j, carry):
    r = j // per_row
    col = (j % per_row) * 16
    ref[r, pl.ds(col, 16)] = vec
    return carry

  lax.fori_loop(0, nrows * per_row, body, 0)


def _sc_scatter_body(width, table_hbm, srci_hbm, dsti_hbm, out_hbm,
                     srci_v, dsti_v, rows, table_sh, acc_sh, sems_g, sems_s):
  c = lax.axis_index("c")
  s = lax.axis_index("s")
  t = c * NS + s

  pltpu.async_copy(srci_hbm.at[t], srci_v, sems_g.at[0])
  pltpu.async_copy(dsti_hbm.at[t], dsti_v, sems_g.at[1])
  pltpu.async_copy(table_hbm.at[pl.ds(s * RPT, RPT)],
                   table_sh.at[pl.ds(s * RPT, RPT)], sems_g.at[2])
  _fill_rows(rows.at[0], K, width, 0.0)
  for j in range(RPT // K):
    pltpu.sync_copy(rows.at[0], acc_sh.at[pl.ds(s * RPT + j * K, K)])
  pltpu.make_async_copy(srci_hbm.at[t], srci_v, sems_g.at[0]).wait()
  pltpu.make_async_copy(dsti_hbm.at[t], dsti_v, sems_g.at[1]).wait()
  pltpu.make_async_copy(table_hbm.at[pl.ds(s * RPT, RPT)],
                        table_sh.at[pl.ds(s * RPT, RPT)], sems_g.at[2]).wait()
  plsc.subcore_barrier()

  def gather(buf, j):
    pltpu.async_copy(table_sh.at[srci_v.at[j]], rows.at[buf], sems_g.at[buf])

  def wait_gather(buf, j):
    pltpu.make_async_copy(table_sh.at[srci_v.at[j]], rows.at[buf],
                          sems_g.at[buf]).wait()

  def scatter(buf, j):
    pltpu.async_copy(rows.at[buf], acc_sh.at[dsti_v.at[j]], sems_s.at[buf],
                     add=True)

  def wait_scatter(buf, j):
    pltpu.make_async_copy(rows.at[buf], acc_sh.at[dsti_v.at[j]],
                          sems_s.at[buf]).wait()

  for b in range(NB):
    gather(b, b)

  def round_(g, carry):
    for half in range(2):
      for b in range(WAVE):
        buf = half * WAVE + b
        j = (2 * g + half) * WAVE + b
        wait_gather(buf, j)
        scatter(buf, j)

      @pl.when(g + 1 < C // NB)
      def _():
        for b in range(WAVE):
          buf = half * WAVE + b
          j = (2 * g + half) * WAVE + b
          wait_scatter(buf, j)
          gather(buf, j + NB)
    return carry

  lax.fori_loop(0, C // NB, round_, 0)
  for buf in range(NB):
    wait_scatter(buf, C - NB + buf)
  plsc.subcore_barrier()

  pltpu.sync_copy(acc_sh.at[pl.ds(s * RPT, RPT)], out_hbm.at[c, pl.ds(s * RPT, RPT)])


def _make_sc_scatter(width):
  mesh = plsc.VectorSubcoreMesh(core_axis_name="c", subcore_axis_name="s")
  return pl.kernel(
      functools.partial(_sc_scatter_body, width),
      out_type=jax.ShapeDtypeStruct((NC, NP, width), jnp.float32),
      mesh=mesh,
      scratch_types=[
          pltpu.VMEM((C, K), jnp.int32),
          pltpu.VMEM((C, K), jnp.int32),
          pltpu.VMEM((NB, K, width), jnp.float32),
          pltpu.VMEM_SHARED((NP, width), jnp.float32),
          pltpu.VMEM_SHARED((NP, width), jnp.float32),
          pltpu.SemaphoreType.DMA((NB,)),
          pltpu.SemaphoreType.DMA((NB,)),
      ],
      compiler_params=pltpu.CompilerParams(use_tc_tiling_on_sc=False),
      name=f"sc_edge_scatter_{width}",
  )


DK = 128
DC = EPT_PAD // DK


def _sc_deg_body(dsti_hbm, out_hbm, dsti_v, ones_v, zbuf_v, acc_sh, sem):
  c = lax.axis_index("c")
  s = lax.axis_index("s")
  t = c * NS + s

  pltpu.async_copy(dsti_hbm.at[t], dsti_v, sem)
  _fill_rows(zbuf_v, DK, 16, 0.0)
  _fill_rows(ones_v, DK, 16, 1.0)
  for j in range(RPT // DK):
    pltpu.sync_copy(zbuf_v, acc_sh.at[pl.ds(s * RPT + j * DK, DK)])
  pltpu.make_async_copy(dsti_hbm.at[t], dsti_v, sem).wait()
  plsc.subcore_barrier()

  def chunk(j, carry):
    pltpu.sync_copy(ones_v, acc_sh.at[dsti_v.at[j]], add=True)
    return carry

  lax.fori_loop(0, DC, chunk, 0)
  plsc.subcore_barrier()
  pltpu.sync_copy(acc_sh.at[pl.ds(s * RPT, RPT)], out_hbm.at[c, pl.ds(s * RPT, RPT)])


def _make_sc_deg():
  mesh = plsc.VectorSubcoreMesh(core_axis_name="c", subcore_axis_name="s")
  return pl.kernel(
      _sc_deg_body,
      out_type=jax.ShapeDtypeStruct((NC, NP, 16), jnp.float32),
      mesh=mesh,
      scratch_types=[
          pltpu.VMEM((DC, DK), jnp.int32),
          pltpu.VMEM((DK, 16), jnp.float32),
          pltpu.VMEM((DK, 16), jnp.float32),
          pltpu.VMEM_SHARED((NP, 16), jnp.float32),
          pltpu.SemaphoreType.DMA,
      ],
      compiler_params=pltpu.CompilerParams(use_tc_tiling_on_sc=False),
      name="sc_degree",
  )


_PREC = lax.Precision.HIGHEST


def _bn(h, g, b, m, v):
  return g * (h - m) / jnp.sqrt(v + 1e-3) + b


def _mm(a, w):
  return jnp.dot(a.astype(jnp.bfloat16), w.astype(jnp.bfloat16),
                 preferred_element_type=jnp.float32)


def _tc1_body(x_ref, deg_ref, w_ref, g_ref, b_ref, m_ref, v_ref, y_ref, rs_ref):
  deg = deg_ref[0] + deg_ref[1] + 1.0
  rs = lax.rsqrt(deg)
  rs_ref[...] = rs
  h0 = _bn(x_ref[...], g_ref[...], b_ref[...], m_ref[...], v_ref[...])
  y_ref[...] = rs[:, 0:1] * _mm(h0, w_ref[...])


def _tc1(xp, degp, w1, g1, b1, m1, v1):
  return pl.pallas_call(
      _tc1_body,
      grid=(NSTEPS,),
      in_specs=[
          pl.BlockSpec((BLK, D), lambda b: (b, 0)),
          pl.BlockSpec((NC, BLK, 16), lambda b: (0, b, 0)),
          pl.BlockSpec((D, H), lambda b: (0, 0)),
          pl.BlockSpec((1, D), lambda b: (0, 0)),
          pl.BlockSpec((1, D), lambda b: (0, 0)),
          pl.BlockSpec((1, D), lambda b: (0, 0)),
          pl.BlockSpec((1, D), lambda b: (0, 0)),
      ],
      out_specs=[
          pl.BlockSpec((BLK, H), lambda b: (b, 0)),
          pl.BlockSpec((BLK, 16), lambda b: (b, 0)),
      ],
      out_shape=[
          jax.ShapeDtypeStruct((NP, H), jnp.float32),
          jax.ShapeDtypeStruct((NP, 16), jnp.float32),
      ],
  )(xp, degp, w1, g1, b1, m1, v1)


def _tc2_body(y1_ref, s_ref, rs_ref, w_ref, bc_ref, g_ref, b_ref, m_ref, v_ref,
              y2_ref):
  r = rs_ref[...][:, 0:1]
  a = r * (y1_ref[...] + s_ref[0] + s_ref[1]) + bc_ref[...]
  h = jnp.maximum(a, 0.0)
  hb = _bn(h, g_ref[...], b_ref[...], m_ref[...], v_ref[...])
  y2_ref[...] = r * _mm(hb, w_ref[...])


def _tc2(y1, sp, rs16, w2, bconv1, g2, b2, m2, v2):
  return pl.pallas_call(
      _tc2_body,
      grid=(NSTEPS,),
      in_specs=[
          pl.BlockSpec((BLK, H), lambda b: (b, 0)),
          pl.BlockSpec((NC, BLK, H), lambda b: (0, b, 0)),
          pl.BlockSpec((BLK, 16), lambda b: (b, 0)),
          pl.BlockSpec((H, H), lambda b: (0, 0)),
          pl.BlockSpec((1, H), lambda b: (0, 0)),
          pl.BlockSpec((1, H), lambda b: (0, 0)),
          pl.BlockSpec((1, H), lambda b: (0, 0)),
          pl.BlockSpec((1, H), lambda b: (0, 0)),
          pl.BlockSpec((1, H), lambda b: (0, 0)),
      ],
      out_specs=pl.BlockSpec((BLK, H), lambda b: (b, 0)),
      out_shape=jax.ShapeDtypeStruct((NP, H), jnp.float32),
  )(y1, sp, rs16, w2, bconv1, g2, b2, m2, v2)


def _tc3_body(y2_ref, s_ref, rs_ref, wfa_ref, cfa_ref, bc_ref, g_ref, b_ref,
              m_ref, v_ref, ip_ref,
              wd1_ref, bd1_ref, wd2_ref, bd2_ref, out_ref, pool_acc):
  step = pl.program_id(0)
  r = rs_ref[...][:, 0:1]
  a = r * (y2_ref[...] + s_ref[0] + s_ref[1]) + bc_ref[...]
  h = jnp.maximum(a, 0.0)
  hb = _bn(h, g_ref[...], b_ref[...], m_ref[...], v_ref[...])
  z = _mm(hb, wfa_ref[...]) + cfa_ref[...]
  p = z[:, :P] * jax.nn.sigmoid(z[:, P:])
  ids = ip_ref[...]
  onehot = (ids == lax.broadcasted_iota(jnp.int32, (G, BLK), 0)).astype(jnp.float32)
  part = jnp.dot(onehot, p, preferred_element_type=jnp.float32, precision=_PREC)

  @pl.when(step == 0)
  def _():
    pool_acc[...] = part

  @pl.when(step > 0)
  def _():
    pool_acc[...] += part

  @pl.when(step == pl.num_programs(0) - 1)
  def _():
    d1 = jnp.maximum(_mm(pool_acc[...], wd1_ref[...]) + bd1_ref[...], 0.0)
    out_ref[...] = _mm(d1, wd2_ref[...]) + bd2_ref[...]


def _tc3(y2, sp, rs16, wfa, cfa, bconv2, g3, b3, m3, v3, ip, wd1, bd1, wd2, bd2):
  return pl.pallas_call(
      _tc3_body,
      grid=(NSTEPS,),
      in_specs=[
          pl.BlockSpec((BLK, H), lambda b: (b, 0)),
          pl.BlockSpec((NC, BLK, H), lambda b: (0, b, 0)),
          pl.BlockSpec((BLK, 16), lambda b: (b, 0)),
          pl.BlockSpec((H, 2 * P), lambda b: (0, 0)),
          pl.BlockSpec((1, 2 * P), lambda b: (0, 0)),
          pl.BlockSpec((1, H), lambda b: (0, 0)),
          pl.BlockSpec((1, H), lambda b: (0, 0)),
          pl.BlockSpec((1, H), lambda b: (0, 0)),
          pl.BlockSpec((1, H), lambda b: (0, 0)),
          pl.BlockSpec((1, H), lambda b: (0, 0)),
          pl.BlockSpec((1, BLK), lambda b: (0, b)),
          pl.BlockSpec((P, H), lambda b: (0, 0)),
          pl.BlockSpec((1, H), lambda b: (0, 0)),
          pl.BlockSpec((H, 1), lambda b: (0, 0)),
          pl.BlockSpec((1, 1), lambda b: (0, 0)),
      ],
      out_specs=pl.BlockSpec((G, 1), lambda b: (0, 0)),
      out_shape=jax.ShapeDtypeStruct((G, 1), jnp.float32),
      scratch_shapes=[pltpu.VMEM((G, P), jnp.float32)],
  )(y2, sp, rs16, wfa, cfa, bconv2, g3, b3, m3, v3, ip, wd1, bd1, wd2, bd2)


def kernel(x, edge_index, i, gamma1, beta1, mean1, var1, W1, b1,
           gamma2, beta2, mean2, var2, W2, b2,
           gamma3, beta3, mean3, var3,
           Wf, bf, Wa, ba, Wd1, bd1, Wd2, bd2):
  f32 = jnp.float32
  wfa = jnp.concatenate([Wf, Wa], axis=1)
  cfa = jnp.concatenate([bf, ba])[None, :]

  src = edge_index[0].astype(jnp.int32).reshape(NTILES, EPT)
  dst = edge_index[1].astype(jnp.int32).reshape(NTILES, EPT)
  srci = jnp.pad(src, ((0, 0), (0, EPT_PAD - EPT))).reshape(NTILES, C, K)
  dsti = jnp.pad(dst, ((0, 0), (0, EPT_PAD - EPT)),
                 constant_values=N).reshape(NTILES, C, K)

  xp = jnp.pad(x.astype(f32), ((0, NP - N), (0, 0)))
  ip = jnp.pad(i.astype(jnp.int32), (0, NP - N), constant_values=G).reshape(1, NP)

  degp = _make_sc_deg()(dsti.reshape(NTILES, DC, DK))
  y1, rs16 = _tc1(xp, degp, W1, gamma1[None, :], beta1[None, :],
                  mean1[None, :], var1[None, :])
  s1p = _make_sc_scatter(H)(y1, srci, dsti)
  y2 = _tc2(y1, s1p, rs16, W2, b1[None, :], gamma2[None, :], beta2[None, :],
            mean2[None, :], var2[None, :])
  s2p = _make_sc_scatter(H)(y2, srci, dsti)
  return _tc3(y2, s2p, rs16, wfa, cfa, b2[None, :], gamma3[None, :],
              beta3[None, :], mean3[None, :], var3[None, :], ip,
              Wd1, bd1[None, :], Wd2, bd2[None, :])
